# baseline (speedup 1.0000x reference)
import functools

import jax
import jax.numpy as jnp
from jax import lax
from jax.experimental import pallas as pl
from jax.experimental.pallas import tpu as pltpu
from jax.experimental.pallas import tpu_sc as plsc

POOL = 7
CELLS = POOL * POOL
KWIN = 6
SCALE = 0.03125
B, C, H, W = 2, 256, 25, 25
N = 1000
NC, NS = 2, 16
CCHUNKS = 2
CCH = C // CCHUNKS
NVREG = CCH // 32
GROUPS = NC * (NS // CCHUNKS)
G_BIG = 63
G_SMALL = 62
N0 = 8 * G_BIG
N1 = N - N0
STAGE = CCH * CELLS


def _roipool_sc():
    mesh = plsc.VectorSubcoreMesh(core_axis_name="c", subcore_axis_name="s")

    @functools.partial(
        pl.kernel,
        out_type=(
            jax.ShapeDtypeStruct((N0 * C * CELLS,), jnp.float32),
            jax.ShapeDtypeStruct((N1 * C * CELLS,), jnp.float32),
        ),
        mesh=mesh,
        compiler_params=pltpu.CompilerParams(needs_layout_passes=False),
        scratch_types=[
            pltpu.VMEM((B * H * W * CCH // 2,), jnp.float32),
            pltpu.VMEM((G_BIG * 32,), jnp.int32),
            pltpu.VMEM((2 * STAGE,), jnp.float32),
            pltpu.SemaphoreType.DMA,
        ],
    )
    def k(feat_hbm, roi_hbm, out0_hbm, out1_hbm, feat_l, roi_l, stage, sem):
        cidx = lax.axis_index("c")
        sidx = lax.axis_index("s")
        chunk = sidx & 1
        grp = cidx * (GROUPS // NC) + (sidx >> 1)
        big = grp < 8
        nroi = jnp.where(big, G_BIG, G_SMALL)
        base = jnp.where(big, grp * G_BIG, N0 + (grp - 8) * G_SMALL)
        base_local = jnp.where(big, grp * G_BIG, (grp - 8) * G_SMALL)

        fs = B * H * W * CCH // 2
        pltpu.sync_copy(
            feat_hbm.at[pl.ds(pl.multiple_of(chunk * fs, 8), fs)], feat_l
        )
        pltpu.sync_copy(
            roi_hbm.at[pl.ds(base * 32, G_BIG * 32)],
            roi_l.at[pl.ds(0, G_BIG * 32)],
        )

        lane2x49 = lax.iota(jnp.int32, 16) * (2 * CELLS)
        neginf = jnp.full((32,), -jnp.inf, jnp.bfloat16)

        def roi_body(i, _):
            va = roi_l[pl.ds(i * 32, 16)]
            vb = roi_l[pl.ds(i * 32 + 16, 16)]
            bat = vb[14]
            pixbase = bat * (H * W)
            buf = (i & 1) * STAGE

            @pl.when(i >= 2)
            def _():
                pltpu.make_async_copy(
                    stage.at[pl.ds(buf, STAGE)],
                    out0_hbm.at[pl.ds(0, STAGE)],
                    sem,
                ).wait()

            for ph in range(POOL):
                hs = va[ph]
                he = va[7 + ph]
                for pw in range(POOL):
                    ws = vb[pw]
                    we = vb[7 + pw]

                    def h_body(h, acc):
                        rowoff = (pixbase + h * W) * (CCH // 2)

                        def w_body(w, acc):
                            off = rowoff + w * (CCH // 2)
                            return tuple(
                                jnp.maximum(
                                    acc[v],
                                    plsc.bitcast(
                                        feat_l[pl.ds(off + v * 16, 16)],
                                        jnp.bfloat16,
                                    ),
                                )
                                for v in range(NVREG)
                            )

                        return lax.fori_loop(ws, we, w_body, acc)

                    acc = lax.fori_loop(hs, he, h_body, (neginf,) * NVREG)
                    empty = (he <= hs) | (we <= ws)
                    cell = ph * POOL + pw
                    for v in range(NVREG):
                        val = jnp.where(empty, jnp.bfloat16(0.0), acc[v])
                        lo, hi = plsc.unpack(
                            val, format=plsc.PackFormat.INTERLEAVED
                        )
                        cbase = buf + v * 32 * CELLS + cell
                        plsc.store_scatter(stage, [lane2x49 + cbase], lo)
                        plsc.store_scatter(
                            stage, [lane2x49 + (cbase + CELLS)], hi
                        )

            out_off = ((base_local + i) * CCHUNKS + chunk) * STAGE

            @pl.when(cidx == 0)
            def _():
                pltpu.async_copy(
                    stage.at[pl.ds(buf, STAGE)],
                    out0_hbm.at[pl.ds(out_off, STAGE)],
                    sem,
                )

            @pl.when(cidx == 1)
            def _():
                pltpu.async_copy(
                    stage.at[pl.ds(buf, STAGE)],
                    out1_hbm.at[pl.ds(out_off, STAGE)],
                    sem,
                )

            return 0

        lax.fori_loop(0, nroi, roi_body, 0)
        for _ in range(2):
            pltpu.make_async_copy(
                stage.at[pl.ds(0, STAGE)],
                out0_hbm.at[pl.ds(0, STAGE)],
                sem,
            ).wait()

    return k


def _bin_tables():
    import numpy as _np

    tf = _np.zeros((32, 8), _np.int32)
    tc = _np.zeros((32, 8), _np.int32)
    recip = _np.float32(1.0) / _np.float32(POOL)
    for rh in range(1, 32):
        b = _np.float32(rh) * recip
        for p in range(8):
            prod = _np.float32(p) * b
            tf[rh, p] = int(_np.floor(prod))
            tc[rh, p] = int(_np.ceil(prod))
    return jnp.asarray(tf.reshape(-1)), jnp.asarray(tc.reshape(-1))


def _bounds(rois, roibatches):
    r = jnp.round(lax.stop_gradient(rois) * SCALE).astype(jnp.int32)
    x1, y1, x2, y2 = r[:, 0], r[:, 1], r[:, 2], r[:, 3]
    roi_w = jnp.maximum(x2 - x1 + 1, 1)
    roi_h = jnp.maximum(y2 - y1 + 1, 1)
    tf, tc = _bin_tables()
    p = jnp.arange(POOL)
    hs = jnp.clip(jnp.take(tf, roi_h[:, None] * 8 + p[None, :]) + y1[:, None], 0, H)
    he = jnp.clip(jnp.take(tc, roi_h[:, None] * 8 + p[None, :] + 1) + y1[:, None], 0, H)
    ws = jnp.clip(jnp.take(tf, roi_w[:, None] * 8 + p[None, :]) + x1[:, None], 0, W)
    we = jnp.clip(jnp.take(tc, roi_w[:, None] * 8 + p[None, :] + 1) + x1[:, None], 0, W)
    he_c = jnp.minimum(he, hs + KWIN)
    we_c = jnp.minimum(we, ws + KWIN)
    z = jnp.zeros((N, 1), jnp.int32)
    pack = jnp.concatenate(
        [hs, he_c, z, z, ws, we_c, roibatches[:, None], z],
        axis=1,
    )
    pack = jnp.concatenate([pack, jnp.zeros((8, 32), jnp.int32)], axis=0)
    return pack.reshape((N + 8) * 32)


def kernel(feat, rois, roibatches):
    feat_r = (
        feat.transpose(0, 2, 3, 1)
        .reshape(B, H, W, CCHUNKS, CCH)
        .transpose(3, 0, 1, 2, 4)
        .reshape(CCHUNKS * B * H * W * CCH)
        .astype(jnp.bfloat16)
    )
    feat_r = jax.lax.bitcast_convert_type(
        feat_r.reshape(-1, 2), jnp.float32
    )
    roi_pack = _bounds(rois, roibatches)
    o0, o1 = _roipool_sc()(feat_r, roi_pack)
    return jnp.concatenate([o0, o1]).reshape(N, C, POOL, POOL)


# speedup vs baseline: 1.0935x; 1.0935x over previous
import functools

import jax
import jax.numpy as jnp
from jax import lax
from jax.experimental import pallas as pl
from jax.experimental.pallas import tpu as pltpu
from jax.experimental.pallas import tpu_sc as plsc

POOL = 7
CELLS = POOL * POOL
KWIN = 6
SCALE = 0.03125
B, C, H, W = 2, 256, 25, 25
N = 1000
NC, NS = 2, 16
CCHUNKS = 2
CCH = C // CCHUNKS
NVREG = CCH // 32
GROUPS = NC * (NS // CCHUNKS)
G_BIG = 63
G_SMALL = 62
NHALF = 4 * G_BIG + 4 * G_SMALL
STAGE = CCH * CELLS


def _roipool_sc():
    mesh = plsc.VectorSubcoreMesh(core_axis_name="c", subcore_axis_name="s")

    @functools.partial(
        pl.kernel,
        out_type=(
            jax.ShapeDtypeStruct((NHALF * C * CELLS,), jnp.float32),
            jax.ShapeDtypeStruct((NHALF * C * CELLS,), jnp.float32),
        ),
        mesh=mesh,
        compiler_params=pltpu.CompilerParams(needs_layout_passes=False),
        scratch_types=[
            pltpu.VMEM((B * H * W * CCH // 2,), jnp.float32),
            pltpu.VMEM((G_BIG * 32,), jnp.int32),
            pltpu.VMEM((2 * STAGE,), jnp.float32),
            pltpu.SemaphoreType.DMA,
        ],
    )
    def k(feat_hbm, roi_hbm, out0_hbm, out1_hbm, feat_l, roi_l, stage, sem):
        cidx = lax.axis_index("c")
        sidx = lax.axis_index("s")
        chunk = sidx & 1
        g8 = sidx >> 1
        big = g8 < 4
        nroi = jnp.where(big, G_BIG, G_SMALL)
        base_local = jnp.where(big, g8 * G_BIG, 4 * G_BIG + (g8 - 4) * G_SMALL)
        base = cidx * NHALF + base_local

        fs = B * H * W * CCH // 2
        pltpu.sync_copy(
            feat_hbm.at[pl.ds(pl.multiple_of(chunk * fs, 8), fs)], feat_l
        )
        pltpu.sync_copy(
            roi_hbm.at[pl.ds(base * 32, G_BIG * 32)],
            roi_l.at[pl.ds(0, G_BIG * 32)],
        )

        lane2x49 = lax.iota(jnp.int32, 16) * (2 * CELLS)
        neginf = jnp.full((32,), -jnp.inf, jnp.bfloat16)

        def roi_body(i, _):
            va = roi_l[pl.ds(i * 32, 16)]
            vb = roi_l[pl.ds(i * 32 + 16, 16)]
            bat = vb[14]
            pixbase = bat * (H * W)
            buf = (i & 1) * STAGE

            @pl.when(i >= 2)
            def _():
                pltpu.make_async_copy(
                    stage.at[pl.ds(buf, STAGE)],
                    out0_hbm.at[pl.ds(0, STAGE)],
                    sem,
                ).wait()

            for ph in range(POOL):
                hs = va[ph]
                he = va[7 + ph]
                for pw in range(POOL):
                    ws = vb[pw]
                    we = vb[7 + pw]

                    def h_body(h, acc):
                        rowoff = (pixbase + h * W) * (CCH // 2)

                        def w_body(w, acc):
                            off = rowoff + w * (CCH // 2)
                            return tuple(
                                jnp.maximum(
                                    acc[v],
                                    plsc.bitcast(
                                        feat_l[pl.ds(off + v * 16, 16)],
                                        jnp.bfloat16,
                                    ),
                                )
                                for v in range(NVREG)
                            )

                        return lax.fori_loop(ws, we, w_body, acc)

                    acc = lax.fori_loop(hs, he, h_body, (neginf,) * NVREG)
                    empty = (he <= hs) | (we <= ws)
                    cell = ph * POOL + pw
                    for v in range(NVREG):
                        val = jnp.where(empty, jnp.bfloat16(0.0), acc[v])
                        lo, hi = plsc.unpack(
                            val, format=plsc.PackFormat.INTERLEAVED
                        )
                        cbase = buf + v * 32 * CELLS + cell
                        plsc.store_scatter(stage, [lane2x49 + cbase], lo)
                        plsc.store_scatter(
                            stage, [lane2x49 + (cbase + CELLS)], hi
                        )

            out_off = ((base_local + i) * CCHUNKS + chunk) * STAGE

            @pl.when(cidx == 0)
            def _():
                pltpu.async_copy(
                    stage.at[pl.ds(buf, STAGE)],
                    out0_hbm.at[pl.ds(out_off, STAGE)],
                    sem,
                )

            @pl.when(cidx == 1)
            def _():
                pltpu.async_copy(
                    stage.at[pl.ds(buf, STAGE)],
                    out1_hbm.at[pl.ds(out_off, STAGE)],
                    sem,
                )

            return 0

        lax.fori_loop(0, nroi, roi_body, 0)
        for _ in range(2):
            pltpu.make_async_copy(
                stage.at[pl.ds(0, STAGE)],
                out0_hbm.at[pl.ds(0, STAGE)],
                sem,
            ).wait()

    return k


def _bin_tables():
    import numpy as _np

    tf = _np.zeros((32, 8), _np.int32)
    tc = _np.zeros((32, 8), _np.int32)
    recip = _np.float32(1.0) / _np.float32(POOL)
    for rh in range(1, 32):
        b = _np.float32(rh) * recip
        for p in range(8):
            prod = _np.float32(p) * b
            tf[rh, p] = int(_np.floor(prod))
            tc[rh, p] = int(_np.ceil(prod))
    return jnp.asarray(tf.reshape(-1)), jnp.asarray(tc.reshape(-1))


def _bounds(rois, roibatches):
    r = jnp.round(lax.stop_gradient(rois) * SCALE).astype(jnp.int32)
    x1, y1, x2, y2 = r[:, 0], r[:, 1], r[:, 2], r[:, 3]
    roi_w = jnp.maximum(x2 - x1 + 1, 1)
    roi_h = jnp.maximum(y2 - y1 + 1, 1)
    tf, tc = _bin_tables()
    p = jnp.arange(POOL)
    hs = jnp.clip(jnp.take(tf, roi_h[:, None] * 8 + p[None, :]) + y1[:, None], 0, H)
    he = jnp.clip(jnp.take(tc, roi_h[:, None] * 8 + p[None, :] + 1) + y1[:, None], 0, H)
    ws = jnp.clip(jnp.take(tf, roi_w[:, None] * 8 + p[None, :]) + x1[:, None], 0, W)
    we = jnp.clip(jnp.take(tc, roi_w[:, None] * 8 + p[None, :] + 1) + x1[:, None], 0, W)
    he_c = jnp.minimum(he, hs + KWIN)
    we_c = jnp.minimum(we, ws + KWIN)
    z = jnp.zeros((N, 1), jnp.int32)
    pack = jnp.concatenate(
        [hs, he_c, z, z, ws, we_c, roibatches[:, None], z],
        axis=1,
    )
    pack = jnp.concatenate([pack, jnp.zeros((8, 32), jnp.int32)], axis=0)
    return pack.reshape((N + 8) * 32)


def kernel(feat, rois, roibatches):
    feat_r = (
        feat.transpose(0, 2, 3, 1)
        .reshape(B, H, W, CCHUNKS, CCH)
        .transpose(3, 0, 1, 2, 4)
        .reshape(CCHUNKS * B * H * W * CCH)
        .astype(jnp.bfloat16)
    )
    feat_r = jax.lax.bitcast_convert_type(
        feat_r.reshape(-1, 2), jnp.float32
    )
    roi_pack = _bounds(rois, roibatches)
    o0, o1 = _roipool_sc()(feat_r, roi_pack)
    return jnp.concatenate([o0, o1]).reshape(N, C, POOL, POOL)


# speedup vs baseline: 1.0941x; 1.0005x over previous
import functools

import jax
import jax.numpy as jnp
from jax import lax
from jax.experimental import pallas as pl
from jax.experimental.pallas import tpu as pltpu
from jax.experimental.pallas import tpu_sc as plsc

POOL = 7
CELLS = POOL * POOL
KWIN = 6
SCALE = 0.03125
B, C, H, W = 2, 256, 25, 25
N = 1000
NC, NS = 2, 16
CCHUNKS = 2
CCH = C // CCHUNKS
NVREG = CCH // 32
GROUPS = NC * (NS // CCHUNKS)
G_BIG = 63
G_SMALL = 62
NHALF = 4 * G_BIG + 4 * G_SMALL
STAGE = CCH * CELLS


def _roipool_sc():
    mesh = plsc.VectorSubcoreMesh(core_axis_name="c", subcore_axis_name="s")

    @functools.partial(
        pl.kernel,
        out_type=(
            jax.ShapeDtypeStruct((NHALF * C * CELLS,), jnp.float32),
            jax.ShapeDtypeStruct((NHALF * C * CELLS,), jnp.float32),
        ),
        mesh=mesh,
        compiler_params=pltpu.CompilerParams(
            needs_layout_passes=False, skip_device_barrier=True
        ),
        scratch_types=[
            pltpu.VMEM((B * H * W * CCH // 2,), jnp.float32),
            pltpu.VMEM((G_BIG * 32,), jnp.int32),
            pltpu.VMEM((2 * STAGE,), jnp.float32),
            pltpu.SemaphoreType.DMA,
        ],
    )
    def k(feat_hbm, roi_hbm, out0_hbm, out1_hbm, feat_l, roi_l, stage, sem):
        cidx = lax.axis_index("c")
        sidx = lax.axis_index("s")
        chunk = sidx & 1
        g8 = sidx >> 1
        big = g8 < 4
        nroi = jnp.where(big, G_BIG, G_SMALL)
        base_local = jnp.where(big, g8 * G_BIG, 4 * G_BIG + (g8 - 4) * G_SMALL)
        base = cidx * NHALF + base_local

        fs = B * H * W * CCH // 2
        pltpu.sync_copy(
            feat_hbm.at[pl.ds(pl.multiple_of(chunk * fs, 8), fs)], feat_l
        )
        pltpu.sync_copy(
            roi_hbm.at[pl.ds(base * 32, G_BIG * 32)],
            roi_l.at[pl.ds(0, G_BIG * 32)],
        )

        lane2x49 = lax.iota(jnp.int32, 16) * (2 * CELLS)
        neginf = jnp.full((32,), -jnp.inf, jnp.bfloat16)

        def roi_body(i, _):
            va = roi_l[pl.ds(i * 32, 16)]
            vb = roi_l[pl.ds(i * 32 + 16, 16)]
            bat = vb[14]
            pixbase = bat * (H * W)
            buf = (i & 1) * STAGE

            @pl.when(i >= 2)
            def _():
                pltpu.make_async_copy(
                    stage.at[pl.ds(buf, STAGE)],
                    out0_hbm.at[pl.ds(0, STAGE)],
                    sem,
                ).wait()

            for ph in range(POOL):
                hs = va[ph]
                he = va[7 + ph]
                for pw in range(POOL):
                    ws = vb[pw]
                    we = vb[7 + pw]

                    def h_body(h, acc):
                        rowoff = (pixbase + h * W) * (CCH // 2)

                        def w_body(w, acc):
                            off = rowoff + w * (CCH // 2)
                            return tuple(
                                jnp.maximum(
                                    acc[v],
                                    plsc.bitcast(
                                        feat_l[pl.ds(off + v * 16, 16)],
                                        jnp.bfloat16,
                                    ),
                                )
                                for v in range(NVREG)
                            )

                        return lax.fori_loop(ws, we, w_body, acc)

                    acc = lax.fori_loop(hs, he, h_body, (neginf,) * NVREG)
                    empty = (he <= hs) | (we <= ws)
                    cell = ph * POOL + pw
                    for v in range(NVREG):
                        val = jnp.where(empty, jnp.bfloat16(0.0), acc[v])
                        lo, hi = plsc.unpack(
                            val, format=plsc.PackFormat.INTERLEAVED
                        )
                        cbase = buf + v * 32 * CELLS + cell
                        plsc.store_scatter(stage, [lane2x49 + cbase], lo)
                        plsc.store_scatter(
                            stage, [lane2x49 + (cbase + CELLS)], hi
                        )

            out_off = ((base_local + i) * CCHUNKS + chunk) * STAGE

            @pl.when(cidx == 0)
            def _():
                pltpu.async_copy(
                    stage.at[pl.ds(buf, STAGE)],
                    out0_hbm.at[pl.ds(out_off, STAGE)],
                    sem,
                )

            @pl.when(cidx == 1)
            def _():
                pltpu.async_copy(
                    stage.at[pl.ds(buf, STAGE)],
                    out1_hbm.at[pl.ds(out_off, STAGE)],
                    sem,
                )

            return 0

        lax.fori_loop(0, nroi, roi_body, 0)
        for _ in range(2):
            pltpu.make_async_copy(
                stage.at[pl.ds(0, STAGE)],
                out0_hbm.at[pl.ds(0, STAGE)],
                sem,
            ).wait()

    return k


def _bin_tables():
    import numpy as _np

    tf = _np.zeros((32, 8), _np.int32)
    tc = _np.zeros((32, 8), _np.int32)
    recip = _np.float32(1.0) / _np.float32(POOL)
    for rh in range(1, 32):
        b = _np.float32(rh) * recip
        for p in range(8):
            prod = _np.float32(p) * b
            tf[rh, p] = int(_np.floor(prod))
            tc[rh, p] = int(_np.ceil(prod))
    return jnp.asarray(tf.reshape(-1)), jnp.asarray(tc.reshape(-1))


def _bounds(rois, roibatches):
    r = jnp.round(lax.stop_gradient(rois) * SCALE).astype(jnp.int32)
    x1, y1, x2, y2 = r[:, 0], r[:, 1], r[:, 2], r[:, 3]
    roi_w = jnp.maximum(x2 - x1 + 1, 1)
    roi_h = jnp.maximum(y2 - y1 + 1, 1)
    tf, tc = _bin_tables()
    p = jnp.arange(POOL)
    hs = jnp.clip(jnp.take(tf, roi_h[:, None] * 8 + p[None, :]) + y1[:, None], 0, H)
    he = jnp.clip(jnp.take(tc, roi_h[:, None] * 8 + p[None, :] + 1) + y1[:, None], 0, H)
    ws = jnp.clip(jnp.take(tf, roi_w[:, None] * 8 + p[None, :]) + x1[:, None], 0, W)
    we = jnp.clip(jnp.take(tc, roi_w[:, None] * 8 + p[None, :] + 1) + x1[:, None], 0, W)
    he_c = jnp.minimum(he, hs + KWIN)
    we_c = jnp.minimum(we, ws + KWIN)
    z = jnp.zeros((N, 1), jnp.int32)
    pack = jnp.concatenate(
        [hs, he_c, z, z, ws, we_c, roibatches[:, None], z],
        axis=1,
    )
    pack = jnp.concatenate([pack, jnp.zeros((8, 32), jnp.int32)], axis=0)
    return pack.reshape((N + 8) * 32)


def kernel(feat, rois, roibatches):
    feat_r = (
        feat.transpose(0, 2, 3, 1)
        .reshape(B, H, W, CCHUNKS, CCH)
        .transpose(3, 0, 1, 2, 4)
        .reshape(CCHUNKS * B * H * W * CCH)
        .astype(jnp.bfloat16)
    )
    feat_r = jax.lax.bitcast_convert_type(
        feat_r.reshape(-1, 2), jnp.float32
    )
    roi_pack = _bounds(rois, roibatches)
    o0, o1 = _roipool_sc()(feat_r, roi_pack)
    return jnp.concatenate([o0, o1]).reshape(N, C, POOL, POOL)
